# Initial kernel scaffold; baseline (speedup 1.0000x reference)
import jax
import jax.numpy as jnp
from jax.experimental import pallas as pl


def kernel(feat, W, b):
    raise NotImplementedError("write your pallas kernel here")


# speedup vs baseline: 26.1115x; 26.1115x over previous
import jax
import jax.numpy as jnp
from jax.experimental import pallas as pl

_N = 4096
_D = 512
_H = 256
_K = 15
_BLK = 256


def _proj_kernel(feat_ref, w_ref, b_ref, z_ref):
    z = jnp.dot(feat_ref[...], w_ref[...],
                preferred_element_type=jnp.float32) + b_ref[...]
    norm = jnp.sqrt(jnp.sum(z * z, axis=-1, keepdims=True))
    z_ref[...] = z / jnp.maximum(norm, 1e-12)


def _sim_kernel(zb_ref, z_ref, out_ref):
    i = pl.program_id(0)
    s = jax.lax.dot_general(zb_ref[...], z_ref[...], (((1,), (1,)), ((), ())),
                            preferred_element_type=jnp.float32)
    m = jnp.max(s, axis=-1, keepdims=True)
    for _ in range(_K - 1):
        m = jnp.max(jnp.where(s < m, s, -jnp.inf), axis=-1, keepdims=True)
    rows = jax.lax.broadcasted_iota(jnp.int32, (_BLK, _N), 0) + i * _BLK
    cols = jax.lax.broadcasted_iota(jnp.int32, (_BLK, _N), 1)
    keep = (s >= m) & (rows != cols)
    out_ref[...] = jnp.where(keep, s, 0.0)


def kernel(feat, W, b):
    z = pl.pallas_call(
        _proj_kernel,
        grid=(_N // _BLK,),
        in_specs=[pl.BlockSpec((_BLK, _D), lambda i: (i, 0)),
                  pl.BlockSpec((_D, _H), lambda i: (0, 0)),
                  pl.BlockSpec((1, _H), lambda i: (0, 0))],
        out_specs=pl.BlockSpec((_BLK, _H), lambda i: (i, 0)),
        out_shape=jax.ShapeDtypeStruct((_N, _H), jnp.float32),
    )(feat, W, b.reshape(1, _H))
    out = pl.pallas_call(
        _sim_kernel,
        grid=(_N // _BLK,),
        in_specs=[pl.BlockSpec((_BLK, _H), lambda i: (i, 0)),
                  pl.BlockSpec((_N, _H), lambda i: (0, 0))],
        out_specs=pl.BlockSpec((_BLK, _N), lambda i: (i, 0)),
        out_shape=jax.ShapeDtypeStruct((_N, _N), jnp.float32),
    )(z, z)
    return out
